# Initial kernel scaffold; baseline (speedup 1.0000x reference)
import jax
import jax.numpy as jnp
from jax.experimental import pallas as pl


def kernel(x, edge_index, batch, W1, b1, W2, b2, W3, b3, W4, b4, Wc1, bc1, Wc2, bc2, Wa, ba):
    raise NotImplementedError("write your pallas kernel here")


# speedup vs baseline: 5.4392x; 5.4392x over previous
import functools

import jax
import jax.numpy as jnp
from jax import lax
from jax.experimental import pallas as pl
from jax.experimental.pallas import tpu as pltpu
from jax.experimental.pallas import tpu_sc as plsc

NC = 2
NS = 16
NW = NC * NS
CHUNK = 128
DEGW = 128


def _wid(c, s):
  return s * NC + c


def _chunked_copy(src_at, dst_at, buf, rows, base_src, base_dst):
  bh = buf.shape[0]
  off = 0
  while off < rows:
    step = min(bh, rows - off)
    pltpu.sync_copy(src_at(base_src + off, step), buf.at[pl.ds(0, step)])
    pltpu.sync_copy(buf.at[pl.ds(0, step)], dst_at(base_dst + off, step))
    off += step


def _sc_deg_body(n_pad, n_chunks_per_w, dst_hbm, ones_hbm, zeros_hbm,
                 degp_hbm, idx1_v, ones_v, zbuf_v, deg_sp):
  c = lax.axis_index("c")
  s = lax.axis_index("s")
  w = _wid(c, s)
  rpt = n_pad // NS

  pltpu.sync_copy(ones_hbm, ones_v)
  pltpu.sync_copy(zeros_hbm, zbuf_v)
  off = 0
  while off < rpt:
    step = min(zbuf_v.shape[0], rpt - off)
    pltpu.sync_copy(zbuf_v.at[pl.ds(0, step)],
                    deg_sp.at[pl.ds(s * rpt + off, step)])
    off += step
  plsc.subcore_barrier()

  def body(j, carry):
    pltpu.sync_copy(dst_hbm.at[w * n_chunks_per_w + j], idx1_v)
    pltpu.sync_copy(ones_v, deg_sp.at[idx1_v], add=True)
    return carry

  lax.fori_loop(0, n_chunks_per_w, body, 0)
  plsc.subcore_barrier()
  _chunked_copy(
      lambda o, sz: deg_sp.at[pl.ds(o, sz)],
      lambda o, sz: degp_hbm.at[c, pl.ds(o, sz)],
      zbuf_v, rpt, s * rpt, s * rpt)


def _sc_scatter_body(n_pad, n_chunks_per_w, hws_hbm, src_hbm, dst_hbm,
                     out_hbm, sidx_v, didx1_v, rows_v, acc_sp, sem):
  c = lax.axis_index("c")
  s = lax.axis_index("s")
  w = _wid(c, s)
  rpt = n_pad // NS

  _chunked_copy(
      lambda o, sz: hws_hbm.at[pl.ds(o, sz)],
      lambda o, sz: acc_sp.at[pl.ds(o, sz)],
      rows_v, rpt, s * rpt, s * rpt)
  plsc.subcore_barrier()

  pltpu.sync_copy(src_hbm.at[pl.ds(w * n_chunks_per_w, n_chunks_per_w)],
                  sidx_v)

  def body(j, carry):
    pltpu.async_copy(hws_hbm.at[sidx_v.at[j]], rows_v, sem).wait()
    pltpu.sync_copy(dst_hbm.at[w * n_chunks_per_w + j], didx1_v)
    pltpu.sync_copy(rows_v, acc_sp.at[didx1_v], add=True)
    return carry

  lax.fori_loop(0, n_chunks_per_w, body, 0)
  plsc.subcore_barrier()
  _chunked_copy(
      lambda o, sz: acc_sp.at[pl.ds(o, sz)],
      lambda o, sz: out_hbm.at[c, pl.ds(o, sz)],
      rows_v, rpt, s * rpt, s * rpt)


def _tc_first_body(degp_ref, x_ref, w_ref, dis_ref, hws_ref):
  deg = degp_ref[0, :, 0:1] + degp_ref[1, :, 0:1] + 1.0
  dis = lax.rsqrt(deg)
  dis_ref[...] = dis
  hws_ref[...] = dis * jnp.dot(x_ref[...], w_ref[...],
                               preferred_element_type=jnp.float32)


def _tc_mid_body(p_ref, hws_ref, dis_ref, b_ref, w_ref, out_ref):
  dis = dis_ref[...]
  h = jnp.maximum(dis * (p_ref[0] + p_ref[1] - hws_ref[...]) + b_ref[...],
                  0.0)
  out_ref[...] = dis * jnp.dot(h, w_ref[...],
                               preferred_element_type=jnp.float32)


def _tc_final_body(p_ref, hws_ref, dis_ref, b_ref, batch_ref, wc1_ref,
                   bc1_ref, wc2_ref, bc2_ref, wa_ref, ba_ref, logits_ref,
                   stab_ref):
  n_pad = hws_ref.shape[0]
  dis = dis_ref[...]
  h = jnp.maximum(dis * (p_ref[0] + p_ref[1] - hws_ref[...]) + b_ref[...],
                  0.0)
  ids = batch_ref[...]
  seg = lax.broadcasted_iota(jnp.int32, (64, n_pad), 0)
  oh = (seg == ids).astype(jnp.float32)
  sums = jnp.dot(oh, h, preferred_element_type=jnp.float32)
  cnt = jnp.dot(oh, jnp.ones((n_pad, 1), jnp.float32),
                preferred_element_type=jnp.float32)
  gemb = sums / jnp.maximum(cnt, 1.0)
  hc = jnp.maximum(
      jnp.dot(gemb, wc1_ref[...], preferred_element_type=jnp.float32)
      + bc1_ref[...], 0.0)
  logits_ref[...] = jnp.dot(hc, wc2_ref[...],
                            preferred_element_type=jnp.float32) + bc2_ref[...]
  za = jnp.dot(gemb, wa_ref[...],
               preferred_element_type=jnp.float32) + ba_ref[...]
  stab_ref[...] = 1.0 / (1.0 + jnp.exp(-za))


def kernel(x, edge_index, batch, W1, b1, W2, b2, W3, b3, W4, b4, Wc1, bc1,
           Wc2, bc2, Wa, ba):
  n, d = x.shape
  e = edge_index.shape[1]
  h_dim = W1.shape[1]
  n_seg = 64

  n_pad = ((n + 1 + NS * 8 - 1) // (NS * 8)) * (NS * 8)
  e_pad = ((e + NW * CHUNK * 8 - 1) // (NW * CHUNK * 8)) * (NW * CHUNK * 8)
  n_chunks = e_pad // CHUNK
  n_chunks_per_w = n_chunks // NW

  src = jnp.concatenate(
      [edge_index[0], jnp.full((e_pad - e,), n, jnp.int32)]).reshape(
          n_chunks, CHUNK)
  dst = jnp.concatenate(
      [edge_index[1], jnp.full((e_pad - e,), n, jnp.int32)]).reshape(
          n_chunks, CHUNK)
  x_p = jnp.pad(x, ((0, n_pad - n), (0, 0)))
  batch_p = jnp.pad(batch, (0, n_pad - n),
                    constant_values=n_seg).reshape(1, n_pad).astype(jnp.int32)

  mesh = plsc.VectorSubcoreMesh(core_axis_name="c", subcore_axis_name="s")

  deg_call = pl.kernel(
      functools.partial(_sc_deg_body, n_pad, n_chunks_per_w),
      out_type=jax.ShapeDtypeStruct((NC, n_pad, DEGW), jnp.float32),
      mesh=mesh,
      scratch_types=[
          pltpu.VMEM((CHUNK,), jnp.int32),
          pltpu.VMEM((CHUNK, DEGW), jnp.float32),
          pltpu.VMEM((CHUNK, DEGW), jnp.float32),
          pltpu.VMEM_SHARED((n_pad, DEGW), jnp.float32),
      ],
  )
  degp = deg_call(dst, jnp.ones((CHUNK, DEGW), jnp.float32),
                  jnp.zeros((CHUNK, DEGW), jnp.float32))

  scatter_call = pl.kernel(
      functools.partial(_sc_scatter_body, n_pad, n_chunks_per_w),
      out_type=jax.ShapeDtypeStruct((NC, n_pad, h_dim), jnp.float32),
      mesh=mesh,
      scratch_types=[
          pltpu.VMEM((n_chunks_per_w, CHUNK), jnp.int32),
          pltpu.VMEM((CHUNK,), jnp.int32),
          pltpu.VMEM((CHUNK, h_dim), jnp.float32),
          pltpu.VMEM_SHARED((n_pad, h_dim), jnp.float32),
          pltpu.SemaphoreType.DMA,
      ],
  )

  dis, hws = pl.pallas_call(
      _tc_first_body,
      out_shape=(
          jax.ShapeDtypeStruct((n_pad, 1), jnp.float32),
          jax.ShapeDtypeStruct((n_pad, d), jnp.float32),
      ),
  )(degp, x_p, W1)

  mid_call = pl.pallas_call(
      _tc_mid_body,
      out_shape=jax.ShapeDtypeStruct((n_pad, h_dim), jnp.float32),
  )

  for (b_prev, w_next) in ((b1, W2), (b2, W3), (b3, W4)):
    p = scatter_call(hws, src, dst)
    hws = mid_call(p, hws, dis, b_prev.reshape(1, h_dim), w_next)

  p = scatter_call(hws, src, dst)

  logits, stab = pl.pallas_call(
      _tc_final_body,
      out_shape=(
          jax.ShapeDtypeStruct((n_seg, Wc2.shape[1]), jnp.float32),
          jax.ShapeDtypeStruct((n_seg, 1), jnp.float32),
      ),
  )(p, hws, dis, b4.reshape(1, h_dim), batch_p, Wc1,
    bc1.reshape(1, h_dim), Wc2, bc2.reshape(1, Wc2.shape[1]), Wa,
    ba.reshape(1, 1))

  return (logits, stab[:, 0])


# speedup vs baseline: 6.3482x; 1.1671x over previous
import functools

import jax
import jax.numpy as jnp
from jax import lax
from jax.experimental import pallas as pl
from jax.experimental.pallas import tpu as pltpu
from jax.experimental.pallas import tpu_sc as plsc

NC = 2
NS = 16
NW = NC * NS
CHUNK = 128
DEGW = 128


def _wid(c, s):
  return s * NC + c


def _chunked_copy(src_at, dst_at, buf, rows, base_src, base_dst):
  bh = buf.shape[0]
  off = 0
  while off < rows:
    step = min(bh, rows - off)
    pltpu.sync_copy(src_at(base_src + off, step), buf.at[pl.ds(0, step)])
    pltpu.sync_copy(buf.at[pl.ds(0, step)], dst_at(base_dst + off, step))
    off += step


def _sc_deg_body(n_pad, n_chunks_per_w, dst_hbm, ones_hbm, zeros_hbm,
                 degp_hbm, idx1_v, ones_v, zbuf_v, deg_sp):
  c = lax.axis_index("c")
  s = lax.axis_index("s")
  w = _wid(c, s)
  rpt = n_pad // NS

  pltpu.sync_copy(ones_hbm, ones_v)
  pltpu.sync_copy(zeros_hbm, zbuf_v)
  off = 0
  while off < rpt:
    step = min(zbuf_v.shape[0], rpt - off)
    pltpu.sync_copy(zbuf_v.at[pl.ds(0, step)],
                    deg_sp.at[pl.ds(s * rpt + off, step)])
    off += step
  plsc.subcore_barrier()

  def body(j, carry):
    pltpu.sync_copy(dst_hbm.at[w * n_chunks_per_w + j], idx1_v)
    pltpu.sync_copy(ones_v, deg_sp.at[idx1_v], add=True)
    return carry

  lax.fori_loop(0, n_chunks_per_w, body, 0)
  plsc.subcore_barrier()
  _chunked_copy(
      lambda o, sz: deg_sp.at[pl.ds(o, sz)],
      lambda o, sz: degp_hbm.at[c, pl.ds(o, sz)],
      zbuf_v, rpt, s * rpt, s * rpt)


def _sc_scatter_body(n_pad, n_chunks_per_w, hws_hbm, src_hbm, dst_hbm,
                     out_hbm, sidx_v, didx_a, didx_b, rows_a, rows_b,
                     acc_sp, gsem_a, gsem_b, isem_a, isem_b):
  c = lax.axis_index("c")
  s = lax.axis_index("s")
  w = _wid(c, s)
  rpt = n_pad // NS
  base = w * n_chunks_per_w
  rows = (rows_a, rows_b)
  didx = (didx_a, didx_b)
  gsem = (gsem_a, gsem_b)
  isem = (isem_a, isem_b)

  _chunked_copy(
      lambda o, sz: hws_hbm.at[pl.ds(o, sz)],
      lambda o, sz: acc_sp.at[pl.ds(o, sz)],
      rows_a, rpt, s * rpt, s * rpt)
  plsc.subcore_barrier()

  pltpu.sync_copy(src_hbm.at[pl.ds(base, n_chunks_per_w)], sidx_v)

  pltpu.async_copy(hws_hbm.at[sidx_v.at[0]], rows[0], gsem[0])
  pltpu.async_copy(dst_hbm.at[base], didx[0], isem[0])

  def step(j, par):
    nxt = jnp.where(j + 1 < n_chunks_per_w, j + 1, 0)
    pltpu.async_copy(hws_hbm.at[sidx_v.at[nxt]], rows[1 - par],
                     gsem[1 - par])
    pltpu.async_copy(dst_hbm.at[base + nxt], didx[1 - par], isem[1 - par])
    pltpu.make_async_copy(hws_hbm.at[sidx_v.at[j]], rows[par],
                          gsem[par]).wait()
    pltpu.make_async_copy(dst_hbm.at[base + j], didx[par],
                          isem[par]).wait()
    pltpu.sync_copy(rows[par], acc_sp.at[didx[par]], add=True)

  def body(t, carry):
    step(2 * t, 0)
    step(2 * t + 1, 1)
    return carry

  lax.fori_loop(0, n_chunks_per_w // 2, body, 0)
  pltpu.make_async_copy(hws_hbm.at[sidx_v.at[0]], rows[0], gsem[0]).wait()
  pltpu.make_async_copy(dst_hbm.at[base], didx[0], isem[0]).wait()
  plsc.subcore_barrier()
  _chunked_copy(
      lambda o, sz: acc_sp.at[pl.ds(o, sz)],
      lambda o, sz: out_hbm.at[c, pl.ds(o, sz)],
      rows_a, rpt, s * rpt, s * rpt)


def _tc_first_body(degp_ref, x_ref, w_ref, dis_ref, hws_ref):
  deg = degp_ref[0, :, 0:1] + degp_ref[1, :, 0:1] + 1.0
  dis = lax.rsqrt(deg)
  dis_ref[...] = dis
  hws_ref[...] = dis * jnp.dot(x_ref[...], w_ref[...],
                               preferred_element_type=jnp.float32)


def _tc_mid_body(p_ref, hws_ref, dis_ref, b_ref, w_ref, out_ref):
  dis = dis_ref[...]
  h = jnp.maximum(dis * (p_ref[0] + p_ref[1] - hws_ref[...]) + b_ref[...],
                  0.0)
  out_ref[...] = dis * jnp.dot(h, w_ref[...],
                               preferred_element_type=jnp.float32)


def _tc_final_body(p_ref, hws_ref, dis_ref, b_ref, batch_ref, wc1_ref,
                   bc1_ref, wc2_ref, bc2_ref, wa_ref, ba_ref, logits_ref,
                   stab_ref):
  n_pad = hws_ref.shape[0]
  dis = dis_ref[...]
  h = jnp.maximum(dis * (p_ref[0] + p_ref[1] - hws_ref[...]) + b_ref[...],
                  0.0)
  ids = batch_ref[...]
  seg = lax.broadcasted_iota(jnp.int32, (64, n_pad), 0)
  oh = (seg == ids).astype(jnp.float32)
  sums = jnp.dot(oh, h, preferred_element_type=jnp.float32)
  cnt = jnp.dot(oh, jnp.ones((n_pad, 1), jnp.float32),
                preferred_element_type=jnp.float32)
  gemb = sums / jnp.maximum(cnt, 1.0)
  hc = jnp.maximum(
      jnp.dot(gemb, wc1_ref[...], preferred_element_type=jnp.float32)
      + bc1_ref[...], 0.0)
  logits_ref[...] = jnp.dot(hc, wc2_ref[...],
                            preferred_element_type=jnp.float32) + bc2_ref[...]
  za = jnp.dot(gemb, wa_ref[...],
               preferred_element_type=jnp.float32) + ba_ref[...]
  stab_ref[...] = 1.0 / (1.0 + jnp.exp(-za))


def kernel(x, edge_index, batch, W1, b1, W2, b2, W3, b3, W4, b4, Wc1, bc1,
           Wc2, bc2, Wa, ba):
  n, d = x.shape
  e = edge_index.shape[1]
  h_dim = W1.shape[1]
  n_seg = 64

  n_pad = ((n + 1 + NS * 8 - 1) // (NS * 8)) * (NS * 8)
  e_pad = ((e + NW * CHUNK * 8 - 1) // (NW * CHUNK * 8)) * (NW * CHUNK * 8)
  n_chunks = e_pad // CHUNK
  n_chunks_per_w = n_chunks // NW

  src = jnp.concatenate(
      [edge_index[0], jnp.full((e_pad - e,), n, jnp.int32)]).reshape(
          n_chunks, CHUNK)
  dst = jnp.concatenate(
      [edge_index[1], jnp.full((e_pad - e,), n, jnp.int32)]).reshape(
          n_chunks, CHUNK)
  x_p = jnp.pad(x, ((0, n_pad - n), (0, 0)))
  batch_p = jnp.pad(batch, (0, n_pad - n),
                    constant_values=n_seg).reshape(1, n_pad).astype(jnp.int32)

  mesh = plsc.VectorSubcoreMesh(core_axis_name="c", subcore_axis_name="s")

  deg_call = pl.kernel(
      functools.partial(_sc_deg_body, n_pad, n_chunks_per_w),
      out_type=jax.ShapeDtypeStruct((NC, n_pad, DEGW), jnp.float32),
      mesh=mesh,
      scratch_types=[
          pltpu.VMEM((CHUNK,), jnp.int32),
          pltpu.VMEM((CHUNK, DEGW), jnp.float32),
          pltpu.VMEM((CHUNK, DEGW), jnp.float32),
          pltpu.VMEM_SHARED((n_pad, DEGW), jnp.float32),
      ],
  )
  degp = deg_call(dst, jnp.ones((CHUNK, DEGW), jnp.float32),
                  jnp.zeros((CHUNK, DEGW), jnp.float32))

  scatter_call = pl.kernel(
      functools.partial(_sc_scatter_body, n_pad, n_chunks_per_w),
      out_type=jax.ShapeDtypeStruct((NC, n_pad, h_dim), jnp.float32),
      mesh=mesh,
      scratch_types=[
          pltpu.VMEM((n_chunks_per_w, CHUNK), jnp.int32),
          pltpu.VMEM((CHUNK,), jnp.int32),
          pltpu.VMEM((CHUNK,), jnp.int32),
          pltpu.VMEM((CHUNK, h_dim), jnp.float32),
          pltpu.VMEM((CHUNK, h_dim), jnp.float32),
          pltpu.VMEM_SHARED((n_pad, h_dim), jnp.float32),
          pltpu.SemaphoreType.DMA,
          pltpu.SemaphoreType.DMA,
          pltpu.SemaphoreType.DMA,
          pltpu.SemaphoreType.DMA,
      ],
  )

  dis, hws = pl.pallas_call(
      _tc_first_body,
      out_shape=(
          jax.ShapeDtypeStruct((n_pad, 1), jnp.float32),
          jax.ShapeDtypeStruct((n_pad, d), jnp.float32),
      ),
  )(degp, x_p, W1)

  mid_call = pl.pallas_call(
      _tc_mid_body,
      out_shape=jax.ShapeDtypeStruct((n_pad, h_dim), jnp.float32),
  )

  for (b_prev, w_next) in ((b1, W2), (b2, W3), (b3, W4)):
    p = scatter_call(hws, src, dst)
    hws = mid_call(p, hws, dis, b_prev.reshape(1, h_dim), w_next)

  p = scatter_call(hws, src, dst)

  logits, stab = pl.pallas_call(
      _tc_final_body,
      out_shape=(
          jax.ShapeDtypeStruct((n_seg, Wc2.shape[1]), jnp.float32),
          jax.ShapeDtypeStruct((n_seg, 1), jnp.float32),
      ),
  )(p, hws, dis, b4.reshape(1, h_dim), batch_p, Wc1,
    bc1.reshape(1, h_dim), Wc2, bc2.reshape(1, Wc2.shape[1]), Wa,
    ba.reshape(1, 1))

  return (logits, stab[:, 0])


# speedup vs baseline: 6.5391x; 1.0301x over previous
import functools

import jax
import jax.numpy as jnp
from jax import lax
from jax.experimental import pallas as pl
from jax.experimental.pallas import tpu as pltpu
from jax.experimental.pallas import tpu_sc as plsc

NC = 2
NS = 16
NW = NC * NS
CHUNK = 128
DEGW = 128
NSLOT = 2


def _wid(c, s):
  return s * NC + c


def _zero_fill(zeros_hbm, buf, acc_sp, base, rows):
  pltpu.sync_copy(zeros_hbm, buf)
  off = 0
  while off < rows:
    step = min(buf.shape[0], rows - off)
    pltpu.sync_copy(buf.at[pl.ds(0, step)], acc_sp.at[pl.ds(base + off, step)])
    off += step


def _drain_out(acc_sp, out_at, buf, base, rows):
  off = 0
  while off < rows:
    step = min(buf.shape[0], rows - off)
    pltpu.sync_copy(acc_sp.at[pl.ds(base + off, step)], buf.at[pl.ds(0, step)])
    pltpu.sync_copy(buf.at[pl.ds(0, step)], out_at(base + off, step))
    off += step


def _sc_deg_body(n_pad, n_chunks_per_w, dst_hbm, ones_hbm, zeros_hbm,
                 degp_hbm, idx1_v, ones_v, zbuf_v, deg_sp):
  c = lax.axis_index("c")
  s = lax.axis_index("s")
  w = _wid(c, s)
  rpt = n_pad // NS

  pltpu.sync_copy(ones_hbm, ones_v)
  _zero_fill(zeros_hbm, zbuf_v, deg_sp, s * rpt, rpt)
  plsc.subcore_barrier()

  def body(j, carry):
    pltpu.sync_copy(dst_hbm.at[w * n_chunks_per_w + j], idx1_v)
    pltpu.sync_copy(ones_v, deg_sp.at[idx1_v], add=True)
    return carry

  lax.fori_loop(0, n_chunks_per_w, body, 0)
  plsc.subcore_barrier()
  _drain_out(deg_sp, lambda o, sz: degp_hbm.at[c, pl.ds(o, sz)],
             zbuf_v, s * rpt, rpt)


def _sc_scatter_body(n_pad, n_chunks_per_w, hws_hbm, src_hbm, dst_hbm,
                     zeros_hbm, out_hbm, sidx_v, d0, d1, r0, r1, acc_sp,
                     g0, g1, i0, i1, s0, s1):
  c = lax.axis_index("c")
  s = lax.axis_index("s")
  w = _wid(c, s)
  rpt = n_pad // NS
  base = w * n_chunks_per_w
  didx = (d0, d1)
  rows = (r0, r1)
  gsem = (g0, g1)
  isem = (i0, i1)
  ssem = (s0, s1)

  _zero_fill(zeros_hbm, r0, acc_sp, s * rpt, rpt)
  plsc.subcore_barrier()

  pltpu.sync_copy(src_hbm.at[pl.ds(base, n_chunks_per_w)], sidx_v)

  def issue_gather(m, q):
    pltpu.async_copy(hws_hbm.at[sidx_v.at[m]], rows[q], gsem[q])
    pltpu.async_copy(dst_hbm.at[base + m], didx[q], isem[q])

  def wait_gather(m, r):
    pltpu.make_async_copy(hws_hbm.at[sidx_v.at[m]], rows[r], gsem[r]).wait()
    pltpu.make_async_copy(dst_hbm.at[base + m], didx[r], isem[r]).wait()

  def issue_scatter(r):
    pltpu.async_copy(rows[r], acc_sp.at[didx[r]], ssem[r], add=True)

  def wait_scatter(r):
    pltpu.make_async_copy(rows[r], acc_sp.at[didx[r]], ssem[r]).wait()

  issue_gather(0, 0)
  wait_gather(0, 0)
  issue_scatter(0)
  issue_gather(1, 1)

  def body(t, carry):
    j0 = 2 * t + 1
    for u in range(2):
      j = j0 + u
      r = (1 + u) % NSLOT
      q = u
      wait_gather(j, r)
      issue_scatter(r)
      wait_scatter(q)
      issue_gather(j + 1, q)
    return carry

  lax.fori_loop(0, (n_chunks_per_w - 2) // 2, body, 0)

  j = n_chunks_per_w - 1
  r = j % NSLOT
  wait_gather(j, r)
  issue_scatter(r)
  for r in range(NSLOT):
    wait_scatter(r)

  plsc.subcore_barrier()
  _drain_out(acc_sp, lambda o, sz: out_hbm.at[c, pl.ds(o, sz)],
             r0, s * rpt, rpt)


def _tc_first_body(degp_ref, x_ref, w_ref, dis_ref, hws_ref):
  deg = degp_ref[0, :, 0:1] + degp_ref[1, :, 0:1] + 1.0
  dis = lax.rsqrt(deg)
  dis_ref[...] = dis
  hws_ref[...] = dis * jnp.dot(x_ref[...], w_ref[...],
                               preferred_element_type=jnp.float32)


def _tc_mid_body(p_ref, hws_ref, dis_ref, b_ref, w_ref, out_ref):
  dis = dis_ref[...]
  h = jnp.maximum(dis * (p_ref[0] + p_ref[1] + hws_ref[...]) + b_ref[...],
                  0.0)
  out_ref[...] = dis * jnp.dot(h, w_ref[...],
                               preferred_element_type=jnp.float32)


def _tc_final_body(p_ref, hws_ref, dis_ref, b_ref, batch_ref, wc1_ref,
                   bc1_ref, wc2_ref, bc2_ref, wa_ref, ba_ref, logits_ref,
                   stab_ref):
  n_pad = hws_ref.shape[0]
  dis = dis_ref[...]
  h = jnp.maximum(dis * (p_ref[0] + p_ref[1] + hws_ref[...]) + b_ref[...],
                  0.0)
  ids = batch_ref[...]
  seg = lax.broadcasted_iota(jnp.int32, (64, n_pad), 0)
  oh = (seg == ids).astype(jnp.float32)
  sums = jnp.dot(oh, h, preferred_element_type=jnp.float32)
  cnt = jnp.dot(oh, jnp.ones((n_pad, 1), jnp.float32),
                preferred_element_type=jnp.float32)
  gemb = sums / jnp.maximum(cnt, 1.0)
  hc = jnp.maximum(
      jnp.dot(gemb, wc1_ref[...], preferred_element_type=jnp.float32)
      + bc1_ref[...], 0.0)
  logits_ref[...] = jnp.dot(hc, wc2_ref[...],
                            preferred_element_type=jnp.float32) + bc2_ref[...]
  za = jnp.dot(gemb, wa_ref[...],
               preferred_element_type=jnp.float32) + ba_ref[...]
  stab_ref[...] = 1.0 / (1.0 + jnp.exp(-za))


def kernel(x, edge_index, batch, W1, b1, W2, b2, W3, b3, W4, b4, Wc1, bc1,
           Wc2, bc2, Wa, ba):
  n, d = x.shape
  e = edge_index.shape[1]
  h_dim = W1.shape[1]
  n_seg = 64

  n_pad = ((n + 1 + NS * 8 - 1) // (NS * 8)) * (NS * 8)
  e_pad = ((e + NW * CHUNK * 8 - 1) // (NW * CHUNK * 8)) * (NW * CHUNK * 8)
  n_chunks = e_pad // CHUNK
  n_chunks_per_w = n_chunks // NW

  src = jnp.concatenate(
      [edge_index[0], jnp.full((e_pad - e,), n, jnp.int32)]).reshape(
          n_chunks, CHUNK)
  dst = jnp.concatenate(
      [edge_index[1], jnp.full((e_pad - e,), n, jnp.int32)]).reshape(
          n_chunks, CHUNK)
  x_p = jnp.pad(x, ((0, n_pad - n), (0, 0)))
  batch_p = jnp.pad(batch, (0, n_pad - n),
                    constant_values=n_seg).reshape(1, n_pad).astype(jnp.int32)
  zeros = jnp.zeros((CHUNK, DEGW), jnp.float32)

  mesh = plsc.VectorSubcoreMesh(core_axis_name="c", subcore_axis_name="s")

  deg_call = pl.kernel(
      functools.partial(_sc_deg_body, n_pad, n_chunks_per_w),
      out_type=jax.ShapeDtypeStruct((NC, n_pad, DEGW), jnp.float32),
      mesh=mesh,
      scratch_types=[
          pltpu.VMEM((CHUNK,), jnp.int32),
          pltpu.VMEM((CHUNK, DEGW), jnp.float32),
          pltpu.VMEM((CHUNK, DEGW), jnp.float32),
          pltpu.VMEM_SHARED((n_pad, DEGW), jnp.float32),
      ],
  )
  degp = deg_call(dst, jnp.ones((CHUNK, DEGW), jnp.float32), zeros)

  scatter_call = pl.kernel(
      functools.partial(_sc_scatter_body, n_pad, n_chunks_per_w),
      out_type=jax.ShapeDtypeStruct((NC, n_pad, h_dim), jnp.float32),
      mesh=mesh,
      scratch_types=(
          [pltpu.VMEM((n_chunks_per_w, CHUNK), jnp.int32)]
          + [pltpu.VMEM((CHUNK,), jnp.int32)] * NSLOT
          + [pltpu.VMEM((CHUNK, h_dim), jnp.float32)] * NSLOT
          + [pltpu.VMEM_SHARED((n_pad, h_dim), jnp.float32)]
          + [pltpu.SemaphoreType.DMA] * (3 * NSLOT)
      ),
  )

  dis, hws = pl.pallas_call(
      _tc_first_body,
      out_shape=(
          jax.ShapeDtypeStruct((n_pad, 1), jnp.float32),
          jax.ShapeDtypeStruct((n_pad, d), jnp.float32),
      ),
  )(degp, x_p, W1)

  mid_call = pl.pallas_call(
      _tc_mid_body,
      out_shape=jax.ShapeDtypeStruct((n_pad, h_dim), jnp.float32),
  )

  for (b_prev, w_next) in ((b1, W2), (b2, W3), (b3, W4)):
    p = scatter_call(hws, src, dst, zeros)
    hws = mid_call(p, hws, dis, b_prev.reshape(1, h_dim), w_next)

  p = scatter_call(hws, src, dst, zeros)

  logits, stab = pl.pallas_call(
      _tc_final_body,
      out_shape=(
          jax.ShapeDtypeStruct((n_seg, Wc2.shape[1]), jnp.float32),
          jax.ShapeDtypeStruct((n_seg, 1), jnp.float32),
      ),
  )(p, hws, dis, b4.reshape(1, h_dim), batch_p, Wc1,
    bc1.reshape(1, h_dim), Wc2, bc2.reshape(1, Wc2.shape[1]), Wa,
    ba.reshape(1, 1))

  return (logits, stab[:, 0])
